# Initial kernel scaffold; baseline (speedup 1.0000x reference)
import jax
import jax.numpy as jnp
from jax.experimental import pallas as pl


def kernel(x, edge_index, batch, pre_w, pre_b, l0_w1, l0_b1, l0_w2, l0_b2, bn0_g, bn0_b, l1_w1, l1_b1, l1_w2, l1_b2, bn1_g, bn1_b, l2_w1, l2_b1, l2_w2, l2_b2, post_w1, post_b1, post_w2, post_b2):
    raise NotImplementedError("write your pallas kernel here")


# speedup vs baseline: 6.8805x; 6.8805x over previous
import functools

import jax
import jax.numpy as jnp
from jax import lax
from jax.experimental import pallas as pl
from jax.experimental.pallas import tpu as pltpu
from jax.experimental.pallas import tpu_sc as plsc

_N = 10000
_F = 128
_E = 320000
_EMBED = 16
_NPAD = 10240
_NC = 2
_NS = 16
_NW = _NC * _NS
_EPW = _E // _NW
_CHUNK = 100
_NCHUNK = _EPW // _CHUNK
_ROWS_PT = _NPAD // _NS


@functools.cache
def _make_sc_agg():
    mesh = plsc.VectorSubcoreMesh(core_axis_name="c", subcore_axis_name="s",
                                  num_cores=_NC, num_subcores=_NS)

    @functools.partial(
        pl.kernel,
        mesh=mesh,
        out_type=jax.ShapeDtypeStruct((_NC, _NPAD, _F), jnp.float32),
        scratch_types=[
            pltpu.VMEM((_NCHUNK, _CHUNK), jnp.int32),
            pltpu.VMEM((_NCHUNK, _CHUNK), jnp.int32),
            pltpu.VMEM((_CHUNK, _F), jnp.float32),
            pltpu.VMEM_SHARED((_NPAD, _F), jnp.float32),
            pltpu.SemaphoreType.DMA,
        ],
    )
    def _sc_agg(h_hbm, src_hbm, dst_hbm, zeros_hbm, out_hbm,
                src_v, dst_v, rows_v, acc_sh, sem):
        cid = lax.axis_index("c")
        sid = lax.axis_index("s")
        wid = sid * _NC + cid
        pltpu.sync_copy(src_hbm.at[wid], src_v)
        pltpu.sync_copy(dst_hbm.at[wid], dst_v)
        r0 = sid * _ROWS_PT
        pltpu.sync_copy(zeros_hbm.at[pl.ds(r0, _ROWS_PT)],
                        acc_sh.at[pl.ds(r0, _ROWS_PT)])
        plsc.subcore_barrier()

        def body(j, carry):
            pltpu.async_copy(h_hbm.at[src_v.at[j]], rows_v, sem).wait()
            pltpu.sync_copy(rows_v, acc_sh.at[dst_v.at[j]], add=True)
            return carry

        lax.fori_loop(0, _NCHUNK, body, 0)
        plsc.subcore_barrier()
        pltpu.sync_copy(acc_sh.at[pl.ds(r0, _ROWS_PT)],
                        out_hbm.at[cid, pl.ds(r0, _ROWS_PT)])

    return _sc_agg


def _leaky(v):
    return jnp.where(v >= 0, v, 0.01 * v)


def _pre_body(x_ref, w_ref, b_ref, o_ref):
    o_ref[...] = jnp.dot(x_ref[...], w_ref[...],
                         preferred_element_type=jnp.float32) + b_ref[...]


def _mlp_stats_body(h_ref, a_ref, w1_ref, b1_ref, w2_ref, b2_ref,
                    y_ref, s_ref):
    z = h_ref[...] + a_ref[0, :_N, :] + a_ref[1, :_N, :]
    t = _leaky(jnp.dot(z, w1_ref[...],
                       preferred_element_type=jnp.float32) + b1_ref[...])
    y = jnp.dot(t, w2_ref[...], preferred_element_type=jnp.float32) + b2_ref[...]
    y_ref[...] = y
    s_ref[0:1, :] = jnp.sum(y, axis=0, keepdims=True)
    s_ref[1:2, :] = jnp.sum(y * y, axis=0, keepdims=True)


def _mlp_body(h_ref, a_ref, w1_ref, b1_ref, w2_ref, b2_ref, y_ref):
    z = h_ref[...] + a_ref[0, :_N, :] + a_ref[1, :_N, :]
    t = _leaky(jnp.dot(z, w1_ref[...],
                       preferred_element_type=jnp.float32) + b1_ref[...])
    y_ref[...] = jnp.dot(t, w2_ref[...],
                         preferred_element_type=jnp.float32) + b2_ref[...]


def _bn_body(y_ref, s_ref, g_ref, b_ref, o_ref):
    m = s_ref[0:1, :] * (1.0 / _N)
    v = s_ref[1:2, :] * (1.0 / _N) - m * m
    scale = lax.rsqrt(v + 1e-5) * g_ref[...]
    o_ref[...] = (y_ref[...] - m) * scale + b_ref[...]


def _post_body(h_ref, w1_ref, b1_ref, w2_ref, b2_ref, o_ref):
    t = _leaky(jnp.dot(h_ref[...], w1_ref[...],
                       preferred_element_type=jnp.float32) + b1_ref[...])
    o_ref[...] = jnp.dot(t, w2_ref[...],
                         preferred_element_type=jnp.float32) + b2_ref[...]


def _tc(body, out_shapes):
    return pl.pallas_call(body, out_shape=out_shapes)


def kernel(x, edge_index, batch, pre_w, pre_b, l0_w1, l0_b1, l0_w2, l0_b2,
           bn0_g, bn0_b, l1_w1, l1_b1, l1_w2, l1_b2, bn1_g, bn1_b,
           l2_w1, l2_b1, l2_w2, l2_b2, post_w1, post_b1, post_w2, post_b2):
    f32 = jnp.float32
    src = edge_index[0].reshape(_NW, _NCHUNK, _CHUNK)
    dst = edge_index[1].reshape(_NW, _NCHUNK, _CHUNK)
    zeros = jnp.zeros((_NPAD, _F), f32)

    h_sd = jax.ShapeDtypeStruct((_N, _F), f32)
    s_sd = jax.ShapeDtypeStruct((2, _F), f32)

    h = _tc(_pre_body, h_sd)(x, pre_w, pre_b.reshape(1, _F))

    layer_params = [
        (l0_w1, l0_b1, l0_w2, l0_b2, bn0_g, bn0_b),
        (l1_w1, l1_b1, l1_w2, l1_b2, bn1_g, bn1_b),
        (l2_w1, l2_b1, l2_w2, l2_b2, None, None),
    ]
    sc_agg = _make_sc_agg()
    for li, (w1, b1, w2, b2, g, b) in enumerate(layer_params):
        agg = sc_agg(h, src, dst, zeros)
        if li < 2:
            y, s = _tc(_mlp_stats_body, (h_sd, s_sd))(
                h, agg, w1, b1.reshape(1, _F), w2, b2.reshape(1, _F))
            h = _tc(_bn_body, h_sd)(y, s, g.reshape(1, _F), b.reshape(1, _F))
        else:
            h = _tc(_mlp_body, h_sd)(
                h, agg, w1, b1.reshape(1, _F), w2, b2.reshape(1, _F))

    out = _tc(_post_body, jax.ShapeDtypeStruct((_N, _EMBED), f32))(
        h, post_w1, post_b1.reshape(1, _F), post_w2, post_b2.reshape(1, _EMBED))
    return out.reshape(_N * _EMBED // 16000, 16000)


# speedup vs baseline: 8.7543x; 1.2723x over previous
import functools

import jax
import jax.numpy as jnp
from jax import lax
from jax.experimental import pallas as pl
from jax.experimental.pallas import tpu as pltpu
from jax.experimental.pallas import tpu_sc as plsc

_N = 10000
_F = 128
_FH = 64
_E = 320000
_EMBED = 16
_NPAD = 10240
_NC = 2
_NS = 16
_EPT = _E // _NS
_CHUNK = 128
_NCHUNK = 158
_EPT_PAD = _NCHUNK * _CHUNK
_ROWS_PT = _NPAD // _NS


@functools.cache
def _make_sc_agg():
    mesh = plsc.VectorSubcoreMesh(core_axis_name="c", subcore_axis_name="s",
                                  num_cores=_NC, num_subcores=_NS)

    @functools.partial(
        pl.kernel,
        mesh=mesh,
        out_type=jax.ShapeDtypeStruct((_NC, _NPAD, _FH), jnp.float32),
        scratch_types=[
            pltpu.VMEM((_NCHUNK, _CHUNK), jnp.int32),
            pltpu.VMEM((_NCHUNK, _CHUNK), jnp.int32),
            pltpu.VMEM((_CHUNK, _FH), jnp.float32),
            pltpu.VMEM((_CHUNK, _FH), jnp.float32),
            pltpu.VMEM_SHARED((_NPAD, _FH), jnp.float32),
            pltpu.SemaphoreType.DMA,
            pltpu.SemaphoreType.DMA,
        ],
        compiler_params=pltpu.CompilerParams(use_tc_tiling_on_sc=False),
    )
    def _sc_agg(h_hbm, src_hbm, dst_hbm, zeros_hbm, out_hbm,
                src_v, dst_v, rows0_v, rows1_v, acc_sh, sem0, sem1):
        cid = lax.axis_index("c")
        sid = lax.axis_index("s")
        hh = h_hbm.at[cid]
        pltpu.sync_copy(src_hbm.at[sid], src_v)
        pltpu.sync_copy(dst_hbm.at[sid], dst_v)
        r0 = sid * _ROWS_PT
        pltpu.sync_copy(zeros_hbm.at[pl.ds(r0, _ROWS_PT)],
                        acc_sh.at[pl.ds(r0, _ROWS_PT)])
        plsc.subcore_barrier()

        pltpu.async_copy(hh.at[src_v.at[0]], rows0_v, sem0)
        pltpu.async_copy(hh.at[src_v.at[1]], rows1_v, sem1)

        def body(jj, carry):
            j = 2 * jj
            pltpu.make_async_copy(hh.at[src_v.at[j]], rows0_v, sem0).wait()
            pltpu.sync_copy(rows0_v, acc_sh.at[dst_v.at[j]], add=True)
            pltpu.async_copy(hh.at[src_v.at[j + 2]], rows0_v, sem0)
            pltpu.make_async_copy(hh.at[src_v.at[j + 1]], rows1_v, sem1).wait()
            pltpu.sync_copy(rows1_v, acc_sh.at[dst_v.at[j + 1]], add=True)
            pltpu.async_copy(hh.at[src_v.at[j + 3]], rows1_v, sem1)
            return carry

        lax.fori_loop(0, _NCHUNK // 2 - 1, body, 0)
        jl = _NCHUNK - 2
        pltpu.make_async_copy(hh.at[src_v.at[jl]], rows0_v, sem0).wait()
        pltpu.sync_copy(rows0_v, acc_sh.at[dst_v.at[jl]], add=True)
        pltpu.make_async_copy(hh.at[src_v.at[jl + 1]], rows1_v, sem1).wait()
        pltpu.sync_copy(rows1_v, acc_sh.at[dst_v.at[jl + 1]], add=True)
        plsc.subcore_barrier()
        pltpu.sync_copy(acc_sh.at[pl.ds(r0, _ROWS_PT)],
                        out_hbm.at[cid, pl.ds(r0, _ROWS_PT)])

    return _sc_agg


def _leaky(v):
    return jnp.where(v >= 0, v, 0.01 * v)


def _split(y, o_ref):
    o_ref[0, :, :] = y[:, :_FH]
    o_ref[1, :, :] = y[:, _FH:]


def _join(h_ref, a_ref):
    return jnp.concatenate(
        [h_ref[0] + a_ref[0, :_N, :], h_ref[1] + a_ref[1, :_N, :]], axis=-1)


def _pre_body(x_ref, w_ref, b_ref, o_ref):
    y = jnp.dot(x_ref[...], w_ref[...],
                preferred_element_type=jnp.float32) + b_ref[...]
    _split(y, o_ref)


def _mlp_stats_body(h_ref, a_ref, w1_ref, b1_ref, w2_ref, b2_ref,
                    y_ref, s_ref):
    z = _join(h_ref, a_ref)
    t = _leaky(jnp.dot(z, w1_ref[...],
                       preferred_element_type=jnp.float32) + b1_ref[...])
    y = jnp.dot(t, w2_ref[...], preferred_element_type=jnp.float32) + b2_ref[...]
    y_ref[...] = y
    s_ref[0:1, :] = jnp.sum(y, axis=0, keepdims=True)
    s_ref[1:2, :] = jnp.sum(y * y, axis=0, keepdims=True)


def _mlp_body(h_ref, a_ref, w1_ref, b1_ref, w2_ref, b2_ref, y_ref):
    z = _join(h_ref, a_ref)
    t = _leaky(jnp.dot(z, w1_ref[...],
                       preferred_element_type=jnp.float32) + b1_ref[...])
    y_ref[...] = jnp.dot(t, w2_ref[...],
                         preferred_element_type=jnp.float32) + b2_ref[...]


def _bn_body(y_ref, s_ref, g_ref, b_ref, o_ref):
    m = s_ref[0:1, :] * (1.0 / _N)
    v = s_ref[1:2, :] * (1.0 / _N) - m * m
    scale = lax.rsqrt(v + 1e-5) * g_ref[...]
    _split((y_ref[...] - m) * scale + b_ref[...], o_ref)


def _post_body(h_ref, w1_ref, b1_ref, w2_ref, b2_ref, o_ref):
    t = _leaky(jnp.dot(h_ref[...], w1_ref[...],
                       preferred_element_type=jnp.float32) + b1_ref[...])
    o_ref[...] = jnp.dot(t, w2_ref[...],
                         preferred_element_type=jnp.float32) + b2_ref[...]


def _tc(body, out_shapes):
    return pl.pallas_call(body, out_shape=out_shapes)


def kernel(x, edge_index, batch, pre_w, pre_b, l0_w1, l0_b1, l0_w2, l0_b2,
           bn0_g, bn0_b, l1_w1, l1_b1, l1_w2, l1_b2, bn1_g, bn1_b,
           l2_w1, l2_b1, l2_w2, l2_b2, post_w1, post_b1, post_w2, post_b2):
    f32 = jnp.float32
    npad_e = _EPT_PAD - _EPT
    pad_src = jnp.broadcast_to((jnp.arange(npad_e, dtype=jnp.int32) * 97) % _N,
                               (_NS, npad_e))
    pad_dst = jnp.broadcast_to(_N + (jnp.arange(npad_e, dtype=jnp.int32)
                                     % (_NPAD - _N)), (_NS, npad_e))
    src = jnp.concatenate([edge_index[0].reshape(_NS, _EPT), pad_src], axis=1)
    dst = jnp.concatenate([edge_index[1].reshape(_NS, _EPT), pad_dst], axis=1)
    src = src.reshape(_NS, _NCHUNK, _CHUNK)
    dst = dst.reshape(_NS, _NCHUNK, _CHUNK)
    zeros = jnp.zeros((_NPAD, _FH), f32)

    hs_sd = jax.ShapeDtypeStruct((_NC, _N, _FH), f32)
    y_sd = jax.ShapeDtypeStruct((_N, _F), f32)
    s_sd = jax.ShapeDtypeStruct((2, _F), f32)

    h = _tc(_pre_body, hs_sd)(x, pre_w, pre_b.reshape(1, _F))

    layer_params = [
        (l0_w1, l0_b1, l0_w2, l0_b2, bn0_g, bn0_b),
        (l1_w1, l1_b1, l1_w2, l1_b2, bn1_g, bn1_b),
        (l2_w1, l2_b1, l2_w2, l2_b2, None, None),
    ]
    sc_agg = _make_sc_agg()
    for li, (w1, b1, w2, b2, g, b) in enumerate(layer_params):
        agg = sc_agg(h, src, dst, zeros)
        if li < 2:
            y, s = _tc(_mlp_stats_body, (y_sd, s_sd))(
                h, agg, w1, b1.reshape(1, _F), w2, b2.reshape(1, _F))
            h = _tc(_bn_body, hs_sd)(y, s, g.reshape(1, _F), b.reshape(1, _F))
        else:
            h = _tc(_mlp_body, y_sd)(
                h, agg, w1, b1.reshape(1, _F), w2, b2.reshape(1, _F))

    out = _tc(_post_body, jax.ShapeDtypeStruct((_N, _EMBED), f32))(
        h, post_w1, post_b1.reshape(1, _F), post_w2, post_b2.reshape(1, _EMBED))
    return out.reshape(_N * _EMBED // 16000, 16000)


# speedup vs baseline: 8.8127x; 1.0067x over previous
import functools

import jax
import jax.numpy as jnp
from jax import lax
from jax.experimental import pallas as pl
from jax.experimental.pallas import tpu as pltpu
from jax.experimental.pallas import tpu_sc as plsc

_N = 10000
_F = 128
_FH = 64
_E = 320000
_EMBED = 16
_NPAD = 10240
_NC = 2
_NS = 16
_EPT = _E // _NS
_CHUNK = 128
_NCHUNK = 158
_EPT_PAD = _NCHUNK * _CHUNK
_ROWS_PT = _NPAD // _NS


@functools.cache
def _make_sc_agg():
    mesh = plsc.VectorSubcoreMesh(core_axis_name="c", subcore_axis_name="s",
                                  num_cores=_NC, num_subcores=_NS)

    @functools.partial(
        pl.kernel,
        mesh=mesh,
        out_type=jax.ShapeDtypeStruct((_NC, _NPAD, _FH), jnp.float32),
        scratch_types=[
            pltpu.VMEM((_NCHUNK, _CHUNK), jnp.int32),
            pltpu.VMEM((_NCHUNK, _CHUNK), jnp.int32),
            pltpu.VMEM((_CHUNK, _FH), jnp.float32),
            pltpu.VMEM((_CHUNK, _FH), jnp.float32),
            pltpu.VMEM_SHARED((_NPAD, _FH), jnp.float32),
            pltpu.SemaphoreType.DMA,
            pltpu.SemaphoreType.DMA,
        ],
        compiler_params=pltpu.CompilerParams(use_tc_tiling_on_sc=False),
    )
    def _sc_agg(h_hbm, src_hbm, dst_hbm, zeros_hbm, out_hbm,
                src_v, dst_v, rows0_v, rows1_v, acc_sh, sem0, sem1):
        cid = lax.axis_index("c")
        sid = lax.axis_index("s")
        hh = h_hbm.at[cid]
        pltpu.sync_copy(src_hbm.at[sid], src_v)
        pltpu.sync_copy(dst_hbm.at[sid], dst_v)
        r0 = sid * _ROWS_PT
        pltpu.sync_copy(zeros_hbm.at[pl.ds(r0, _ROWS_PT)],
                        acc_sh.at[pl.ds(r0, _ROWS_PT)])
        plsc.subcore_barrier()

        pltpu.async_copy(hh.at[src_v.at[0]], rows0_v, sem0)
        pltpu.async_copy(hh.at[src_v.at[1]], rows1_v, sem1)

        def body(jj, carry):
            j = 2 * jj
            pltpu.make_async_copy(hh.at[src_v.at[j]], rows0_v, sem0).wait()
            pltpu.sync_copy(rows0_v, acc_sh.at[dst_v.at[j]], add=True)
            pltpu.async_copy(hh.at[src_v.at[j + 2]], rows0_v, sem0)
            pltpu.make_async_copy(hh.at[src_v.at[j + 1]], rows1_v, sem1).wait()
            pltpu.sync_copy(rows1_v, acc_sh.at[dst_v.at[j + 1]], add=True)
            pltpu.async_copy(hh.at[src_v.at[j + 3]], rows1_v, sem1)
            return carry

        lax.fori_loop(0, _NCHUNK // 2 - 1, body, 0)
        jl = _NCHUNK - 2
        pltpu.make_async_copy(hh.at[src_v.at[jl]], rows0_v, sem0).wait()
        pltpu.sync_copy(rows0_v, acc_sh.at[dst_v.at[jl]], add=True)
        pltpu.make_async_copy(hh.at[src_v.at[jl + 1]], rows1_v, sem1).wait()
        pltpu.sync_copy(rows1_v, acc_sh.at[dst_v.at[jl + 1]], add=True)
        plsc.subcore_barrier()
        pltpu.sync_copy(acc_sh.at[pl.ds(r0, _ROWS_PT)],
                        out_hbm.at[cid, pl.ds(r0, _ROWS_PT)])

    return _sc_agg


_RB = 2000
_NGRID = _N // _RB


def _leaky(v):
    return jnp.where(v >= 0, v, 0.01 * v)


def _split(y, o_ref):
    o_ref[0, :, :] = y[:, :_FH]
    o_ref[1, :, :] = y[:, _FH:]


def _join(h_ref, a_ref):
    return jnp.concatenate(
        [h_ref[0] + a_ref[0], h_ref[1] + a_ref[1]], axis=-1)


def _pre_body(x_ref, w_ref, b_ref, o_ref):
    y = jnp.dot(x_ref[...], w_ref[...],
                preferred_element_type=jnp.float32) + b_ref[...]
    _split(y, o_ref)


def _mlp_stats_body(h_ref, a_ref, w1_ref, b1_ref, w2_ref, b2_ref,
                    y_ref, s_ref):
    z = _join(h_ref, a_ref)
    t = _leaky(jnp.dot(z, w1_ref[...],
                       preferred_element_type=jnp.float32) + b1_ref[...])
    y = jnp.dot(t, w2_ref[...], preferred_element_type=jnp.float32) + b2_ref[...]
    y_ref[...] = y

    @pl.when(pl.program_id(0) == 0)
    def _():
        s_ref[...] = jnp.zeros_like(s_ref)

    s_ref[0:1, :] += jnp.sum(y, axis=0, keepdims=True)
    s_ref[1:2, :] += jnp.sum(y * y, axis=0, keepdims=True)


def _mlp_post_body(h_ref, a_ref, w1_ref, b1_ref, w2_ref, b2_ref,
                   pw1_ref, pb1_ref, pw2_ref, pb2_ref, o_ref):
    z = _join(h_ref, a_ref)
    t = _leaky(jnp.dot(z, w1_ref[...],
                       preferred_element_type=jnp.float32) + b1_ref[...])
    y = jnp.dot(t, w2_ref[...], preferred_element_type=jnp.float32) + b2_ref[...]
    t2 = _leaky(jnp.dot(y, pw1_ref[...],
                        preferred_element_type=jnp.float32) + pb1_ref[...])
    o_ref[...] = jnp.dot(t2, pw2_ref[...],
                         preferred_element_type=jnp.float32) + pb2_ref[...]


def _bn_body(y_ref, s_ref, g_ref, b_ref, o_ref):
    m = s_ref[0:1, :] * (1.0 / _N)
    v = s_ref[1:2, :] * (1.0 / _N) - m * m
    scale = lax.rsqrt(v + 1e-5) * g_ref[...]
    _split((y_ref[...] - m) * scale + b_ref[...], o_ref)


def _full(shape):
    ndim = len(shape)
    return pl.BlockSpec(shape, lambda i, _n=ndim: (0,) * _n)


_BS_H = pl.BlockSpec((_NC, _RB, _FH), lambda i: (0, i, 0))
_BS_Y = pl.BlockSpec((_RB, _F), lambda i: (i, 0))


def kernel(x, edge_index, batch, pre_w, pre_b, l0_w1, l0_b1, l0_w2, l0_b2,
           bn0_g, bn0_b, l1_w1, l1_b1, l1_w2, l1_b2, bn1_g, bn1_b,
           l2_w1, l2_b1, l2_w2, l2_b2, post_w1, post_b1, post_w2, post_b2):
    f32 = jnp.float32
    npad_e = _EPT_PAD - _EPT
    pad_src = jnp.broadcast_to((jnp.arange(npad_e, dtype=jnp.int32) * 97) % _N,
                               (_NS, npad_e))
    pad_dst = jnp.broadcast_to(_N + (jnp.arange(npad_e, dtype=jnp.int32)
                                     % (_NPAD - _N)), (_NS, npad_e))
    src = jnp.concatenate([edge_index[0].reshape(_NS, _EPT), pad_src], axis=1)
    dst = jnp.concatenate([edge_index[1].reshape(_NS, _EPT), pad_dst], axis=1)
    src = src.reshape(_NS, _NCHUNK, _CHUNK)
    dst = dst.reshape(_NS, _NCHUNK, _CHUNK)
    zeros = jnp.zeros((_NPAD, _FH), f32)

    hs_sd = jax.ShapeDtypeStruct((_NC, _N, _FH), f32)
    y_sd = jax.ShapeDtypeStruct((_N, _F), f32)
    s_sd = jax.ShapeDtypeStruct((2, _F), f32)

    wspec = _full((_F, _F))
    bspec = _full((1, _F))

    h = pl.pallas_call(
        _pre_body,
        grid=(_NGRID,),
        in_specs=[_BS_Y, wspec, bspec],
        out_specs=_BS_H,
        out_shape=hs_sd,
    )(x, pre_w, pre_b.reshape(1, _F))

    layer_params = [
        (l0_w1, l0_b1, l0_w2, l0_b2, bn0_g, bn0_b),
        (l1_w1, l1_b1, l1_w2, l1_b2, bn1_g, bn1_b),
        (l2_w1, l2_b1, l2_w2, l2_b2, None, None),
    ]
    sc_agg = _make_sc_agg()
    for li, (w1, b1, w2, b2, g, b) in enumerate(layer_params):
        agg = sc_agg(h, src, dst, zeros)
        if li < 2:
            y, s = pl.pallas_call(
                _mlp_stats_body,
                grid=(_NGRID,),
                in_specs=[_BS_H, _BS_H, wspec, bspec, wspec, bspec],
                out_specs=(_BS_Y, _full((2, _F))),
                out_shape=(y_sd, s_sd),
            )(h, agg, w1, b1.reshape(1, _F), w2, b2.reshape(1, _F))
            h = pl.pallas_call(
                _bn_body,
                grid=(_NGRID,),
                in_specs=[_BS_Y, _full((2, _F)), bspec, bspec],
                out_specs=_BS_H,
                out_shape=hs_sd,
            )(y, s, g.reshape(1, _F), b.reshape(1, _F))
        else:
            out = pl.pallas_call(
                _mlp_post_body,
                grid=(_NGRID,),
                in_specs=[_BS_H, _BS_H, wspec, bspec, wspec, bspec,
                          wspec, bspec, _full((_F, _EMBED)), _full((1, _EMBED))],
                out_specs=pl.BlockSpec((_RB, _EMBED), lambda i: (i, 0)),
                out_shape=jax.ShapeDtypeStruct((_N, _EMBED), f32),
            )(h, agg, w1, b1.reshape(1, _F), w2, b2.reshape(1, _F),
              post_w1, post_b1.reshape(1, _F), post_w2,
              post_b2.reshape(1, _EMBED))

    return out.reshape(_N * _EMBED // 16000, 16000)


# speedup vs baseline: 10.6494x; 1.2084x over previous
import functools

import jax
import jax.numpy as jnp
from jax import lax
from jax.experimental import pallas as pl
from jax.experimental.pallas import tpu as pltpu
from jax.experimental.pallas import tpu_sc as plsc

_N = 10000
_F = 128
_FH = 64
_E = 320000
_EMBED = 16
_NPAD = 10240
_NC = 2
_NS = 16
_EPT = _E // _NS
_CHUNK = 128
_NBUF = 4
_NCHUNK = 160
_EPT_PAD = _NCHUNK * _CHUNK
_ROWS_PT = _NPAD // _NS


@functools.cache
def _make_sc_agg():
    mesh = plsc.VectorSubcoreMesh(core_axis_name="c", subcore_axis_name="s",
                                  num_cores=_NC, num_subcores=_NS)

    @functools.partial(
        pl.kernel,
        mesh=mesh,
        out_type=jax.ShapeDtypeStruct((_NC, _NPAD, _FH), jnp.float32),
        scratch_types=[
            pltpu.VMEM((_NCHUNK, _CHUNK), jnp.int32),
            pltpu.VMEM((_NCHUNK, _CHUNK), jnp.int32),
            [pltpu.VMEM((_CHUNK, _FH), jnp.float32) for _ in range(_NBUF)],
            pltpu.VMEM_SHARED((_NPAD, _FH), jnp.float32),
            [pltpu.SemaphoreType.DMA for _ in range(_NBUF)],
        ],
        compiler_params=pltpu.CompilerParams(use_tc_tiling_on_sc=False),
    )
    def _sc_agg(h_hbm, src_hbm, dst_hbm, zeros_hbm, out_hbm,
                src_v, dst_v, rows_v, acc_sh, sems):
        cid = lax.axis_index("c")
        sid = lax.axis_index("s")
        hh = h_hbm.at[cid]
        pltpu.sync_copy(src_hbm.at[sid], src_v)
        pltpu.sync_copy(dst_hbm.at[sid], dst_v)
        r0 = sid * _ROWS_PT
        pltpu.sync_copy(zeros_hbm.at[pl.ds(r0, _ROWS_PT)],
                        acc_sh.at[pl.ds(r0, _ROWS_PT)])
        plsc.subcore_barrier()

        for b in range(_NBUF):
            pltpu.async_copy(hh.at[src_v.at[b]], rows_v[b], sems[b])

        def body(jj, carry):
            j = _NBUF * jj
            for b in range(_NBUF):
                pltpu.make_async_copy(hh.at[src_v.at[j + b]],
                                      rows_v[b], sems[b]).wait()
                pltpu.sync_copy(rows_v[b], acc_sh.at[dst_v.at[j + b]], add=True)
                pltpu.async_copy(hh.at[src_v.at[j + b + _NBUF]],
                                 rows_v[b], sems[b])
            return carry

        lax.fori_loop(0, _NCHUNK // _NBUF - 1, body, 0)
        jl = _NCHUNK - _NBUF
        for b in range(_NBUF):
            pltpu.make_async_copy(hh.at[src_v.at[jl + b]],
                                  rows_v[b], sems[b]).wait()
            pltpu.sync_copy(rows_v[b], acc_sh.at[dst_v.at[jl + b]], add=True)
        plsc.subcore_barrier()
        pltpu.sync_copy(acc_sh.at[pl.ds(r0, _ROWS_PT)],
                        out_hbm.at[cid, pl.ds(r0, _ROWS_PT)])

    return _sc_agg


_RB = 2000
_NGRID = _N // _RB


def _leaky(v):
    return jnp.where(v >= 0, v, 0.01 * v)


def _split(y, o_ref):
    o_ref[0, :, :] = y[:, :_FH]
    o_ref[1, :, :] = y[:, _FH:]


def _join(h_ref, a_ref):
    return jnp.concatenate(
        [h_ref[0] + a_ref[0], h_ref[1] + a_ref[1]], axis=-1)


def _pre_body(x_ref, w_ref, b_ref, o_ref):
    y = jnp.dot(x_ref[...], w_ref[...],
                preferred_element_type=jnp.float32) + b_ref[...]
    _split(y, o_ref)


def _mlp_stats_body(h_ref, a_ref, w1_ref, b1_ref, w2_ref, b2_ref,
                    y_ref, s_ref):
    z = _join(h_ref, a_ref)
    t = _leaky(jnp.dot(z, w1_ref[...],
                       preferred_element_type=jnp.float32) + b1_ref[...])
    y = jnp.dot(t, w2_ref[...], preferred_element_type=jnp.float32) + b2_ref[...]
    y_ref[...] = y

    @pl.when(pl.program_id(0) == 0)
    def _():
        s_ref[...] = jnp.zeros_like(s_ref)

    s_ref[0:1, :] += jnp.sum(y, axis=0, keepdims=True)
    s_ref[1:2, :] += jnp.sum(y * y, axis=0, keepdims=True)


def _mlp_post_body(h_ref, a_ref, w1_ref, b1_ref, w2_ref, b2_ref,
                   pw1_ref, pb1_ref, pw2_ref, pb2_ref, o_ref):
    z = _join(h_ref, a_ref)
    t = _leaky(jnp.dot(z, w1_ref[...],
                       preferred_element_type=jnp.float32) + b1_ref[...])
    y = jnp.dot(t, w2_ref[...], preferred_element_type=jnp.float32) + b2_ref[...]
    t2 = _leaky(jnp.dot(y, pw1_ref[...],
                        preferred_element_type=jnp.float32) + pb1_ref[...])
    o_ref[...] = jnp.dot(t2, pw2_ref[...],
                         preferred_element_type=jnp.float32) + pb2_ref[...]


def _bn_body(y_ref, s_ref, g_ref, b_ref, o_ref):
    m = s_ref[0:1, :] * (1.0 / _N)
    v = s_ref[1:2, :] * (1.0 / _N) - m * m
    scale = lax.rsqrt(v + 1e-5) * g_ref[...]
    _split((y_ref[...] - m) * scale + b_ref[...], o_ref)


def _full(shape):
    ndim = len(shape)
    return pl.BlockSpec(shape, lambda i, _n=ndim: (0,) * _n)


_BS_H = pl.BlockSpec((_NC, _RB, _FH), lambda i: (0, i, 0))
_BS_Y = pl.BlockSpec((_RB, _F), lambda i: (i, 0))


def kernel(x, edge_index, batch, pre_w, pre_b, l0_w1, l0_b1, l0_w2, l0_b2,
           bn0_g, bn0_b, l1_w1, l1_b1, l1_w2, l1_b2, bn1_g, bn1_b,
           l2_w1, l2_b1, l2_w2, l2_b2, post_w1, post_b1, post_w2, post_b2):
    f32 = jnp.float32
    npad_e = _EPT_PAD - _EPT
    pad_src = jnp.broadcast_to((jnp.arange(npad_e, dtype=jnp.int32) * 97) % _N,
                               (_NS, npad_e))
    pad_dst = jnp.broadcast_to(_N + (jnp.arange(npad_e, dtype=jnp.int32)
                                     % (_NPAD - _N)), (_NS, npad_e))
    src = jnp.concatenate([edge_index[0].reshape(_NS, _EPT), pad_src], axis=1)
    dst = jnp.concatenate([edge_index[1].reshape(_NS, _EPT), pad_dst], axis=1)
    src = src.reshape(_NS, _NCHUNK, _CHUNK)
    dst = dst.reshape(_NS, _NCHUNK, _CHUNK)
    zeros = jnp.zeros((_NPAD, _FH), f32)

    hs_sd = jax.ShapeDtypeStruct((_NC, _N, _FH), f32)
    y_sd = jax.ShapeDtypeStruct((_N, _F), f32)
    s_sd = jax.ShapeDtypeStruct((2, _F), f32)

    wspec = _full((_F, _F))
    bspec = _full((1, _F))

    h = pl.pallas_call(
        _pre_body,
        grid=(_NGRID,),
        in_specs=[_BS_Y, wspec, bspec],
        out_specs=_BS_H,
        out_shape=hs_sd,
    )(x, pre_w, pre_b.reshape(1, _F))

    layer_params = [
        (l0_w1, l0_b1, l0_w2, l0_b2, bn0_g, bn0_b),
        (l1_w1, l1_b1, l1_w2, l1_b2, bn1_g, bn1_b),
        (l2_w1, l2_b1, l2_w2, l2_b2, None, None),
    ]
    sc_agg = _make_sc_agg()
    for li, (w1, b1, w2, b2, g, b) in enumerate(layer_params):
        agg = sc_agg(h, src, dst, zeros)
        if li < 2:
            y, s = pl.pallas_call(
                _mlp_stats_body,
                grid=(_NGRID,),
                in_specs=[_BS_H, _BS_H, wspec, bspec, wspec, bspec],
                out_specs=(_BS_Y, _full((2, _F))),
                out_shape=(y_sd, s_sd),
            )(h, agg, w1, b1.reshape(1, _F), w2, b2.reshape(1, _F))
            h = pl.pallas_call(
                _bn_body,
                grid=(_NGRID,),
                in_specs=[_BS_Y, _full((2, _F)), bspec, bspec],
                out_specs=_BS_H,
                out_shape=hs_sd,
            )(y, s, g.reshape(1, _F), b.reshape(1, _F))
        else:
            out = pl.pallas_call(
                _mlp_post_body,
                grid=(_NGRID,),
                in_specs=[_BS_H, _BS_H, wspec, bspec, wspec, bspec,
                          wspec, bspec, _full((_F, _EMBED)), _full((1, _EMBED))],
                out_specs=pl.BlockSpec((_RB, _EMBED), lambda i: (i, 0)),
                out_shape=jax.ShapeDtypeStruct((_N, _EMBED), f32),
            )(h, agg, w1, b1.reshape(1, _F), w2, b2.reshape(1, _F),
              post_w1, post_b1.reshape(1, _F), post_w2,
              post_b2.reshape(1, _EMBED))

    return out.reshape(_N * _EMBED // 16000, 16000)


# speedup vs baseline: 10.6573x; 1.0007x over previous
import functools

import jax
import jax.numpy as jnp
from jax import lax
from jax.experimental import pallas as pl
from jax.experimental.pallas import tpu as pltpu
from jax.experimental.pallas import tpu_sc as plsc

_N = 10000
_F = 128
_FH = 64
_E = 320000
_EMBED = 16
_NPAD = 10240
_NC = 2
_NS = 16
_EPT = _E // _NS
_CHUNK = 128
_NBUF = 5
_NCHUNK = 160
_EPT_PAD = _NCHUNK * _CHUNK
_ROWS_PT = _NPAD // _NS


@functools.cache
def _make_sc_agg():
    mesh = plsc.VectorSubcoreMesh(core_axis_name="c", subcore_axis_name="s",
                                  num_cores=_NC, num_subcores=_NS)

    @functools.partial(
        pl.kernel,
        mesh=mesh,
        out_type=jax.ShapeDtypeStruct((_NC, _NPAD, _FH), jnp.float32),
        scratch_types=[
            pltpu.VMEM((_NCHUNK, _CHUNK), jnp.int32),
            pltpu.VMEM((_NCHUNK, _CHUNK), jnp.int32),
            [pltpu.VMEM((_CHUNK, _FH), jnp.float32) for _ in range(_NBUF)],
            pltpu.VMEM_SHARED((_NPAD, _FH), jnp.float32),
            [pltpu.SemaphoreType.DMA for _ in range(_NBUF)],
        ],
        compiler_params=pltpu.CompilerParams(use_tc_tiling_on_sc=False),
    )
    def _sc_agg(h_hbm, src_hbm, dst_hbm, zeros_hbm, out_hbm,
                src_v, dst_v, rows_v, acc_sh, sems):
        cid = lax.axis_index("c")
        sid = lax.axis_index("s")
        hh = h_hbm.at[cid]
        pltpu.sync_copy(src_hbm.at[sid], src_v)
        pltpu.sync_copy(dst_hbm.at[sid], dst_v)
        r0 = sid * _ROWS_PT
        pltpu.sync_copy(zeros_hbm.at[pl.ds(r0, _ROWS_PT)],
                        acc_sh.at[pl.ds(r0, _ROWS_PT)])
        plsc.subcore_barrier()

        for b in range(_NBUF):
            pltpu.async_copy(hh.at[src_v.at[b]], rows_v[b], sems[b])

        def body(jj, carry):
            j = _NBUF * jj
            for b in range(_NBUF):
                pltpu.make_async_copy(hh.at[src_v.at[j + b]],
                                      rows_v[b], sems[b]).wait()
                pltpu.sync_copy(rows_v[b], acc_sh.at[dst_v.at[j + b]], add=True)
                pltpu.async_copy(hh.at[src_v.at[j + b + _NBUF]],
                                 rows_v[b], sems[b])
            return carry

        lax.fori_loop(0, _NCHUNK // _NBUF - 1, body, 0)
        jl = _NCHUNK - _NBUF
        for b in range(_NBUF):
            pltpu.make_async_copy(hh.at[src_v.at[jl + b]],
                                  rows_v[b], sems[b]).wait()
            pltpu.sync_copy(rows_v[b], acc_sh.at[dst_v.at[jl + b]], add=True)
        plsc.subcore_barrier()
        pltpu.sync_copy(acc_sh.at[pl.ds(r0, _ROWS_PT)],
                        out_hbm.at[cid, pl.ds(r0, _ROWS_PT)])

    return _sc_agg


_RB = 2000
_NGRID = _N // _RB


def _leaky(v):
    return jnp.where(v >= 0, v, 0.01 * v)


def _split(y, o_ref):
    o_ref[0, :, :] = y[:, :_FH]
    o_ref[1, :, :] = y[:, _FH:]


def _join(h_ref, a_ref):
    return jnp.concatenate(
        [h_ref[0] + a_ref[0], h_ref[1] + a_ref[1]], axis=-1)


def _pre_body(x_ref, w_ref, b_ref, o_ref):
    y = jnp.dot(x_ref[...], w_ref[...],
                preferred_element_type=jnp.float32) + b_ref[...]
    _split(y, o_ref)


def _mlp_stats_body(h_ref, a_ref, w1_ref, b1_ref, w2_ref, b2_ref,
                    y_ref, s_ref):
    z = _join(h_ref, a_ref)
    t = _leaky(jnp.dot(z, w1_ref[...],
                       preferred_element_type=jnp.float32) + b1_ref[...])
    y = jnp.dot(t, w2_ref[...], preferred_element_type=jnp.float32) + b2_ref[...]
    y_ref[...] = y

    @pl.when(pl.program_id(0) == 0)
    def _():
        s_ref[...] = jnp.zeros_like(s_ref)

    s_ref[0:1, :] += jnp.sum(y, axis=0, keepdims=True)
    s_ref[1:2, :] += jnp.sum(y * y, axis=0, keepdims=True)


def _mlp_post_body(h_ref, a_ref, w1_ref, b1_ref, w2_ref, b2_ref,
                   pw1_ref, pb1_ref, pw2_ref, pb2_ref, o_ref):
    z = _join(h_ref, a_ref)
    t = _leaky(jnp.dot(z, w1_ref[...],
                       preferred_element_type=jnp.float32) + b1_ref[...])
    y = jnp.dot(t, w2_ref[...], preferred_element_type=jnp.float32) + b2_ref[...]
    t2 = _leaky(jnp.dot(y, pw1_ref[...],
                        preferred_element_type=jnp.float32) + pb1_ref[...])
    o_ref[...] = jnp.dot(t2, pw2_ref[...],
                         preferred_element_type=jnp.float32) + pb2_ref[...]


def _bn_body(y_ref, s_ref, g_ref, b_ref, o_ref):
    m = s_ref[0:1, :] * (1.0 / _N)
    v = s_ref[1:2, :] * (1.0 / _N) - m * m
    scale = lax.rsqrt(v + 1e-5) * g_ref[...]
    _split((y_ref[...] - m) * scale + b_ref[...], o_ref)


def _full(shape):
    ndim = len(shape)
    return pl.BlockSpec(shape, lambda i, _n=ndim: (0,) * _n)


_BS_H = pl.BlockSpec((_NC, _RB, _FH), lambda i: (0, i, 0))
_BS_Y = pl.BlockSpec((_RB, _F), lambda i: (i, 0))


def kernel(x, edge_index, batch, pre_w, pre_b, l0_w1, l0_b1, l0_w2, l0_b2,
           bn0_g, bn0_b, l1_w1, l1_b1, l1_w2, l1_b2, bn1_g, bn1_b,
           l2_w1, l2_b1, l2_w2, l2_b2, post_w1, post_b1, post_w2, post_b2):
    f32 = jnp.float32
    npad_e = _EPT_PAD - _EPT
    pad_src = jnp.broadcast_to((jnp.arange(npad_e, dtype=jnp.int32) * 97) % _N,
                               (_NS, npad_e))
    pad_dst = jnp.broadcast_to(_N + (jnp.arange(npad_e, dtype=jnp.int32)
                                     % (_NPAD - _N)), (_NS, npad_e))
    src = jnp.concatenate([edge_index[0].reshape(_NS, _EPT), pad_src], axis=1)
    dst = jnp.concatenate([edge_index[1].reshape(_NS, _EPT), pad_dst], axis=1)
    src = src.reshape(_NS, _NCHUNK, _CHUNK)
    dst = dst.reshape(_NS, _NCHUNK, _CHUNK)
    zeros = jnp.zeros((_NPAD, _FH), f32)

    hs_sd = jax.ShapeDtypeStruct((_NC, _N, _FH), f32)
    y_sd = jax.ShapeDtypeStruct((_N, _F), f32)
    s_sd = jax.ShapeDtypeStruct((2, _F), f32)

    wspec = _full((_F, _F))
    bspec = _full((1, _F))

    h = pl.pallas_call(
        _pre_body,
        grid=(_NGRID,),
        in_specs=[_BS_Y, wspec, bspec],
        out_specs=_BS_H,
        out_shape=hs_sd,
    )(x, pre_w, pre_b.reshape(1, _F))

    layer_params = [
        (l0_w1, l0_b1, l0_w2, l0_b2, bn0_g, bn0_b),
        (l1_w1, l1_b1, l1_w2, l1_b2, bn1_g, bn1_b),
        (l2_w1, l2_b1, l2_w2, l2_b2, None, None),
    ]
    sc_agg = _make_sc_agg()
    for li, (w1, b1, w2, b2, g, b) in enumerate(layer_params):
        agg = sc_agg(h, src, dst, zeros)
        if li < 2:
            y, s = pl.pallas_call(
                _mlp_stats_body,
                grid=(_NGRID,),
                in_specs=[_BS_H, _BS_H, wspec, bspec, wspec, bspec],
                out_specs=(_BS_Y, _full((2, _F))),
                out_shape=(y_sd, s_sd),
            )(h, agg, w1, b1.reshape(1, _F), w2, b2.reshape(1, _F))
            h = pl.pallas_call(
                _bn_body,
                grid=(_NGRID,),
                in_specs=[_BS_Y, _full((2, _F)), bspec, bspec],
                out_specs=_BS_H,
                out_shape=hs_sd,
            )(y, s, g.reshape(1, _F), b.reshape(1, _F))
        else:
            out = pl.pallas_call(
                _mlp_post_body,
                grid=(_NGRID,),
                in_specs=[_BS_H, _BS_H, wspec, bspec, wspec, bspec,
                          wspec, bspec, _full((_F, _EMBED)), _full((1, _EMBED))],
                out_specs=pl.BlockSpec((_RB, _EMBED), lambda i: (i, 0)),
                out_shape=jax.ShapeDtypeStruct((_N, _EMBED), f32),
            )(h, agg, w1, b1.reshape(1, _F), w2, b2.reshape(1, _F),
              post_w1, post_b1.reshape(1, _F), post_w2,
              post_b2.reshape(1, _EMBED))

    return out.reshape(_N * _EMBED // 16000, 16000)
